# baseline (speedup 1.0000x reference)
import jax
import jax.numpy as jnp
from jax.experimental import pallas as pl

_NUM_MODELS = 64
_BLK = 16384


def _mlp_body(xa_ref, xb_ref, w0_ref, b0_ref, w1_ref, b1_ref, w2_ref, b2_ref,
              w3_ref, b3_ref, w4_ref, b4_ref, y_ref):
    bf = jnp.bfloat16
    x = jnp.concatenate([xa_ref[...], xb_ref[...]], axis=0).astype(bf)
    h = jnp.dot(w0_ref[...], x, preferred_element_type=jnp.float32)
    h = jnp.maximum(h.astype(bf) + b0_ref[...], 0.0)
    h = jnp.dot(w1_ref[...], h, preferred_element_type=jnp.float32)
    h = jnp.maximum(h.astype(bf) + b1_ref[...], 0.0)
    h = jnp.dot(w2_ref[...], h, preferred_element_type=jnp.float32)
    h = jnp.maximum(h.astype(bf) + b2_ref[...], 0.0)
    h = jnp.dot(w3_ref[...], h, preferred_element_type=jnp.float32)
    h = jnp.maximum(h.astype(bf) + b3_ref[...], 0.0)
    y_ref[...] = (jnp.dot(w4_ref[...], h, preferred_element_type=jnp.float32)
                  + b4_ref[...])


def _diag2(w):
    m, n = w.shape
    z = jnp.zeros((m, n), w.dtype)
    return jnp.concatenate(
        [jnp.concatenate([w, z], axis=1),
         jnp.concatenate([z, w], axis=1)], axis=0)


def kernel(inputs, lods, W0, b0, W1, b1, W2, b2, W3, b3, W4, b4):
    bsz, in_f = inputs.shape
    hid = W1.shape[-1]
    out_f = W4.shape[-1]
    n_lods = int(lods.shape[0])
    half = bsz // 2
    nblk = half // _BLK
    grid = (nblk,)

    bf = jnp.bfloat16
    xT = inputs.T
    w0t = _diag2(W0[0].T).astype(bf)
    w1t = _diag2(W1[0].T).astype(bf)
    w2t = _diag2(W2[0].T).astype(bf)
    w3t = _diag2(W3[0].T).astype(bf)
    w4t = _diag2(W4[0].T).astype(bf)
    b0c = jnp.tile(b0[0], 2)[:, None].astype(bf)
    b1c = jnp.tile(b1[0], 2)[:, None].astype(bf)
    b2c = jnp.tile(b2[0], 2)[:, None].astype(bf)
    b3c = jnp.tile(b3[0], 2)[:, None].astype(bf)
    b4c = jnp.tile(b4[0], 2)[:, None]

    full = lambda shape: pl.BlockSpec(shape, lambda i: (0,) * len(shape))
    y2 = pl.pallas_call(
        _mlp_body,
        grid=grid,
        in_specs=[
            pl.BlockSpec((in_f, _BLK), lambda i: (0, i)),
            pl.BlockSpec((in_f, _BLK), lambda i, _n=nblk: (0, i + _n)),
            full((2 * hid, 2 * in_f)), full((2 * hid, 1)),
            full((2 * hid, 2 * hid)), full((2 * hid, 1)),
            full((2 * hid, 2 * hid)), full((2 * hid, 1)),
            full((2 * hid, 2 * hid)), full((2 * hid, 1)),
            full((2 * out_f, 2 * hid)), full((2 * out_f, 1)),
        ],
        out_specs=pl.BlockSpec((2 * out_f, _BLK), lambda i: (0, i)),
        out_shape=jax.ShapeDtypeStruct((2 * out_f, half), jnp.float32),
    )(xT, xT, w0t, b0c, w1t, b1c, w2t, b2c, w3t, b3c, w4t, b4c)

    yT = jnp.concatenate([y2[:out_f], y2[out_f:]], axis=1)
    y = yT.T
    model_outputs = jnp.broadcast_to(y[:, None, :], (bsz, n_lods, out_f))
    sel_idx = jnp.zeros((bsz,), jnp.int32)
    logit_row = jnp.concatenate(
        [jnp.zeros((1,), inputs.dtype),
         jnp.full((_NUM_MODELS - 1,), -999.9, inputs.dtype)])
    logits = jnp.broadcast_to(logit_row[None, :], (bsz, _NUM_MODELS))
    prob_row = jnp.concatenate(
        [jnp.ones((1,), inputs.dtype),
         jnp.zeros((_NUM_MODELS - 1,), inputs.dtype)])
    probs = jnp.broadcast_to(prob_row[None, :], (bsz, _NUM_MODELS))
    return (model_outputs, sel_idx, logits, probs)


# speedup vs baseline: 1.0880x; 1.0880x over previous
import jax
import jax.numpy as jnp
from jax.experimental import pallas as pl

_NUM_MODELS = 64
_BLK = 32768


def _mlp_body(x_ref, w0_ref, b0_ref, w1_ref, b1_ref, w2_ref, b2_ref,
              w3_ref, b3_ref, w4_ref, b4_ref, y_ref):
    bf = jnp.bfloat16
    x = x_ref[...].astype(bf)
    h = jnp.dot(w0_ref[...], x, preferred_element_type=jnp.float32)
    h = jnp.maximum(h.astype(bf) + b0_ref[...], 0.0)
    h = jnp.dot(w1_ref[...], h, preferred_element_type=jnp.float32)
    h = jnp.maximum(h.astype(bf) + b1_ref[...], 0.0)
    h = jnp.dot(w2_ref[...], h, preferred_element_type=jnp.float32)
    h = jnp.maximum(h.astype(bf) + b2_ref[...], 0.0)
    h = jnp.dot(w3_ref[...], h, preferred_element_type=jnp.float32)
    h = jnp.maximum(h.astype(bf) + b3_ref[...], 0.0)
    y_ref[...] = (jnp.dot(w4_ref[...], h, preferred_element_type=jnp.float32)
                  + b4_ref[...])


def kernel(inputs, lods, W0, b0, W1, b1, W2, b2, W3, b3, W4, b4):
    bsz, in_f = inputs.shape
    hid = W1.shape[-1]
    out_f = W4.shape[-1]
    n_lods = int(lods.shape[0])
    grid = (bsz // _BLK,)

    bf = jnp.bfloat16
    w0t = W0[0].T.astype(bf)
    w1t, w2t, w3t = (W1[0].T.astype(bf), W2[0].T.astype(bf),
                     W3[0].T.astype(bf))
    w4t = W4[0].T.astype(bf)
    b0c = b0[0][:, None].astype(bf)
    b1c, b2c, b3c = (b1[0][:, None].astype(bf), b2[0][:, None].astype(bf),
                     b3[0][:, None].astype(bf))
    b4c = b4[0][:, None]

    full = lambda shape: pl.BlockSpec(shape, lambda i: (0,) * len(shape))
    yT = pl.pallas_call(
        _mlp_body,
        grid=grid,
        in_specs=[
            pl.BlockSpec((in_f, _BLK), lambda i: (0, i)),
            full((hid, in_f)), full((hid, 1)),
            full((hid, hid)), full((hid, 1)),
            full((hid, hid)), full((hid, 1)),
            full((hid, hid)), full((hid, 1)),
            full((out_f, hid)), full((out_f, 1)),
        ],
        out_specs=pl.BlockSpec((out_f, _BLK), lambda i: (0, i)),
        out_shape=jax.ShapeDtypeStruct((out_f, bsz), jnp.float32),
    )(inputs.T, w0t, b0c, w1t, b1c, w2t, b2c, w3t, b3c, w4t, b4c)

    y = yT.T
    model_outputs = jnp.broadcast_to(y[:, None, :], (bsz, n_lods, out_f))
    sel_idx = jnp.zeros((bsz,), jnp.int32)
    logit_row = jnp.concatenate(
        [jnp.zeros((1,), inputs.dtype),
         jnp.full((_NUM_MODELS - 1,), -999.9, inputs.dtype)])
    logits = jnp.broadcast_to(logit_row[None, :], (bsz, _NUM_MODELS))
    prob_row = jnp.concatenate(
        [jnp.ones((1,), inputs.dtype),
         jnp.zeros((_NUM_MODELS - 1,), inputs.dtype)])
    probs = jnp.broadcast_to(prob_row[None, :], (bsz, _NUM_MODELS))
    return (model_outputs, sel_idx, logits, probs)
